# Initial kernel scaffold; baseline (speedup 1.0000x reference)
import jax
import jax.numpy as jnp
from jax.experimental import pallas as pl


def kernel(direct_effects, edge_index, W):
    raise NotImplementedError("write your pallas kernel here")


# speedup vs baseline: 4.4802x; 4.4802x over previous
import functools

import jax
import jax.numpy as jnp
from jax import lax
from jax.experimental import pallas as pl
from jax.experimental.pallas import tpu as pltpu
from jax.experimental.pallas import tpu_sc as plsc

N_GENES = 10000
N_EDGES = 320000
BATCH = 128
K_STEPS = 3

N_PAD = 10240
NW = 32
EPW = N_EDGES // NW
CHUNK = 80
NCHUNK = EPW // CHUNK
ROWS_PER_TILE = N_PAD // 16
INIT_CHUNK = 128
LANES = 16
VREGS_PER_ROW = BATCH // LANES

FLAT = N_PAD * BATCH
FPW = FLAT // NW
CCH = 8192

_mesh = plsc.VectorSubcoreMesh(core_axis_name="c", subcore_axis_name="s")


@functools.partial(
    pl.kernel,
    out_type=jax.ShapeDtypeStruct((2, N_PAD, BATCH), jnp.float32),
    mesh=_mesh,
    scratch_types=[
        pltpu.VMEM_SHARED((N_PAD, BATCH), jnp.float32),
        pltpu.VMEM((NCHUNK, 1, CHUNK), jnp.int32),
        pltpu.VMEM((NCHUNK, 1, CHUNK), jnp.int32),
        pltpu.VMEM((1, CHUNK * LANES), jnp.float32),
        pltpu.VMEM((CHUNK, BATCH), jnp.float32),
        pltpu.SemaphoreType.DMA,
    ],
)
def _scatter_step(p_hbm, srcR, dstR, wR, out_hbm,
                  acc, src_v, dst_v, w_v, rows, gsem):
    cid = lax.axis_index("c")
    sid = lax.axis_index("s")
    wid = cid * 16 + sid
    row0 = sid * ROWS_PER_TILE

    zero16f = jnp.zeros((LANES,), jnp.float32)

    def zero_row(r, c2):
        for j in range(VREGS_PER_ROW):
            rows[r, pl.ds(j * LANES, LANES)] = zero16f
        return c2

    lax.fori_loop(0, CHUNK, zero_row, 0)

    @pl.when(cid == 0)
    def _():
        def cp(k, c2):
            base = row0 + k * INIT_CHUNK
            pltpu.sync_copy(p_hbm.at[pl.ds(base, INIT_CHUNK)],
                            acc.at[pl.ds(base, INIT_CHUNK)])
            return c2

        lax.fori_loop(0, ROWS_PER_TILE // INIT_CHUNK, cp, 0)

    @pl.when(cid == 1)
    def _():
        def zf(k, c2):
            base = row0 + k * CHUNK
            pltpu.sync_copy(rows, acc.at[pl.ds(base, CHUNK)])
            return c2

        lax.fori_loop(0, ROWS_PER_TILE // CHUNK, zf, 0)

    plsc.subcore_barrier()

    pltpu.sync_copy(srcR.at[wid], src_v)
    pltpu.sync_copy(dstR.at[wid], dst_v)

    def chunk_step(ci, carry):
        gcopy = pltpu.async_copy(p_hbm.at[src_v.at[ci, 0]], rows, gsem)
        pltpu.sync_copy(wR.at[wid, ci], w_v)
        gcopy.wait()

        def group_step(g, c2):
            for lane in range(LANES):
                e = g * LANES + lane
                wb = w_v[0, pl.ds(e * LANES, LANES)]
                for j in range(VREGS_PER_ROW):
                    sl = pl.ds(j * LANES, LANES)
                    rows[e, sl] = rows[e, sl] * wb
            return c2

        lax.fori_loop(0, CHUNK // LANES, group_step, 0)
        pltpu.sync_copy(rows, acc.at[dst_v.at[ci, 0]], add=True)
        return carry

    lax.fori_loop(0, NCHUNK, chunk_step, 0)

    plsc.subcore_barrier()
    pltpu.sync_copy(acc.at[pl.ds(row0, ROWS_PER_TILE)],
                    out_hbm.at[cid, pl.ds(row0, ROWS_PER_TILE)])


@functools.partial(
    pl.kernel,
    out_type=jax.ShapeDtypeStruct((FLAT,), jnp.float32),
    mesh=_mesh,
    scratch_types=[
        pltpu.VMEM((CCH,), jnp.float32),
        pltpu.VMEM((CCH,), jnp.float32),
    ],
)
def _combine(parts, out, v0, v1):
    cid = lax.axis_index("c")
    sid = lax.axis_index("s")
    wid = cid * 16 + sid
    base = wid * FPW

    def step(k, carry):
        off = base + k * CCH
        pltpu.sync_copy(parts.at[0, pl.ds(off, CCH)], v0)
        pltpu.sync_copy(parts.at[1, pl.ds(off, CCH)], v1)

        def add_step(i, c2):
            sl = pl.ds(i * LANES, LANES)
            v0[sl] = v0[sl] + v1[sl]
            return c2

        lax.fori_loop(0, CCH // LANES, add_step, 0)
        pltpu.sync_copy(v0, out.at[pl.ds(off, CCH)])
        return carry

    lax.fori_loop(0, FPW // CCH, step, 0)


def kernel(direct_effects, edge_index, W):
    x = direct_effects.astype(jnp.float32)
    src = edge_index[0].astype(jnp.int32).reshape(NW, NCHUNK, 1, CHUNK)
    dst = edge_index[1].astype(jnp.int32).reshape(NW, NCHUNK, 1, CHUNK)
    w3 = W.astype(jnp.float32).reshape(NW, NCHUNK, CHUNK)
    wts = jnp.broadcast_to(w3[..., None], (NW, NCHUNK, CHUNK, LANES))
    wts = wts.reshape(NW, NCHUNK, 1, CHUNK * LANES)

    p = jnp.pad(x.T, ((0, N_PAD - N_GENES), (0, 0)))
    for _ in range(K_STEPS):
        parts = _scatter_step(p, src, dst, wts)
        p = _combine(parts.reshape(2, FLAT)).reshape(N_PAD, BATCH)
    return p[:N_GENES].T


# speedup vs baseline: 5.1611x; 1.1520x over previous
import functools

import jax
import jax.numpy as jnp
from jax import lax
from jax.experimental import pallas as pl
from jax.experimental.pallas import tpu as pltpu
from jax.experimental.pallas import tpu_sc as plsc

N_GENES = 10000
N_EDGES = 320000
BATCH = 128
K_STEPS = 3

N_PAD = 10240
NW = 32
CHUNK = 80
NCHUNK = 126
EPW = NCHUNK * CHUNK
E_PAD = NW * EPW
ROWS_PER_TILE = N_PAD // 16
INIT_CHUNK = 128
LANES = 16
VREGS_PER_ROW = BATCH // LANES
NSLOT = 3

FLAT = N_PAD * BATCH
FPW = FLAT // NW
CCH = 8192

_mesh = plsc.VectorSubcoreMesh(core_axis_name="c", subcore_axis_name="s")


@functools.partial(
    pl.kernel,
    out_type=jax.ShapeDtypeStruct((2, N_PAD, BATCH), jnp.float32),
    mesh=_mesh,
    scratch_types=[
        pltpu.VMEM_SHARED((N_PAD, BATCH), jnp.float32),
        pltpu.VMEM((NSLOT, 1, CHUNK), jnp.int32),
        pltpu.VMEM((NSLOT, 1, CHUNK), jnp.int32),
        pltpu.VMEM((NSLOT, 1, CHUNK * LANES), jnp.float32),
        pltpu.VMEM((NSLOT, CHUNK, BATCH), jnp.float32),
        pltpu.SemaphoreType.DMA, pltpu.SemaphoreType.DMA, pltpu.SemaphoreType.DMA,
        pltpu.SemaphoreType.DMA, pltpu.SemaphoreType.DMA, pltpu.SemaphoreType.DMA,
        pltpu.SemaphoreType.DMA, pltpu.SemaphoreType.DMA, pltpu.SemaphoreType.DMA,
    ],
)
def _scatter_step(p_hbm, srcR, dstR, wR, out_hbm,
                  acc, srcb, dstb, wbuf, rows,
                  isem0, isem1, isem2, gsem0, gsem1, gsem2, ssem0, ssem1, ssem2):
    isem = (isem0, isem1, isem2)
    gsem = (gsem0, gsem1, gsem2)
    ssem = (ssem0, ssem1, ssem2)

    cid = lax.axis_index("c")
    sid = lax.axis_index("s")
    wid = cid * 16 + sid
    row0 = sid * ROWS_PER_TILE

    zero16f = jnp.zeros((LANES,), jnp.float32)

    def zero_row(r, c2):
        for j in range(VREGS_PER_ROW):
            rows[0, r, pl.ds(j * LANES, LANES)] = zero16f
        return c2

    lax.fori_loop(0, CHUNK, zero_row, 0)

    @pl.when(cid == 0)
    def _():
        def cp(k, c2):
            base = row0 + k * INIT_CHUNK
            pltpu.sync_copy(p_hbm.at[pl.ds(base, INIT_CHUNK)],
                            acc.at[pl.ds(base, INIT_CHUNK)])
            return c2

        lax.fori_loop(0, ROWS_PER_TILE // INIT_CHUNK, cp, 0)

    @pl.when(cid == 1)
    def _():
        def zf(k, c2):
            base = row0 + k * CHUNK
            pltpu.sync_copy(rows.at[0], acc.at[pl.ds(base, CHUNK)])
            return c2

        lax.fori_loop(0, ROWS_PER_TILE // CHUNK, zf, 0)

    plsc.subcore_barrier()

    def load_idx(ci, s):
        pltpu.async_copy(srcR.at[wid, ci], srcb.at[s], isem[s])

    def wait_idx(ci, s):
        pltpu.make_async_copy(srcR.at[wid, ci], srcb.at[s], isem[s]).wait()

    def issue_gather(ci, s):
        pltpu.async_copy(p_hbm.at[srcb.at[s, 0]], rows.at[s], gsem[s])
        pltpu.async_copy(wR.at[wid, ci], wbuf.at[s], gsem[s])
        pltpu.async_copy(dstR.at[wid, ci], dstb.at[s], gsem[s])

    def wait_gather(ci, s):
        pltpu.make_async_copy(p_hbm.at[srcb.at[s, 0]], rows.at[s], gsem[s]).wait()
        pltpu.make_async_copy(wR.at[wid, ci], wbuf.at[s], gsem[s]).wait()
        pltpu.make_async_copy(dstR.at[wid, ci], dstb.at[s], gsem[s]).wait()

    def issue_scatter(s):
        pltpu.async_copy(rows.at[s], acc.at[dstb.at[s, 0]], ssem[s], add=True)

    def wait_scatter(s):
        pltpu.make_async_copy(rows.at[s], acc.at[dstb.at[s, 0]], ssem[s]).wait()

    def multiply(s):
        def group_step(g, c2):
            for lane in range(LANES):
                e = g * LANES + lane
                wb = wbuf[s, 0, pl.ds(e * LANES, LANES)]
                for j in range(VREGS_PER_ROW):
                    sl = pl.ds(j * LANES, LANES)
                    rows[s, e, sl] = rows[s, e, sl] * wb
            return c2

        lax.fori_loop(0, CHUNK // LANES, group_step, 0)

    load_idx(0, 0)
    load_idx(1, 1)
    load_idx(2, 2)
    wait_idx(0, 0)
    issue_gather(0, 0)
    wait_idx(1, 1)
    issue_gather(1, 1)

    def group(g, carry):
        for b in range(NSLOT):
            ci = g * NSLOT + b
            wait_gather(ci, b)

            @pl.when(ci + NSLOT < NCHUNK)
            def _():
                load_idx(ci + NSLOT, b)

            multiply(b)
            issue_scatter(b)

            s2 = (b + 2) % NSLOT

            @pl.when(ci + 2 < NCHUNK)
            def _():
                wait_idx(ci + 2, s2)

                @pl.when(ci >= 1)
                def _():
                    wait_scatter(s2)

                issue_gather(ci + 2, s2)
        return carry

    lax.fori_loop(0, NCHUNK // NSLOT, group, 0)

    wait_scatter(0)
    wait_scatter(1)
    wait_scatter(2)

    plsc.subcore_barrier()
    pltpu.sync_copy(acc.at[pl.ds(row0, ROWS_PER_TILE)],
                    out_hbm.at[cid, pl.ds(row0, ROWS_PER_TILE)])


@functools.partial(
    pl.kernel,
    out_type=jax.ShapeDtypeStruct((FLAT,), jnp.float32),
    mesh=_mesh,
    scratch_types=[
        pltpu.VMEM((CCH,), jnp.float32),
        pltpu.VMEM((CCH,), jnp.float32),
    ],
)
def _combine(parts, out, v0, v1):
    cid = lax.axis_index("c")
    sid = lax.axis_index("s")
    wid = cid * 16 + sid
    base = wid * FPW

    def step(k, carry):
        off = base + k * CCH
        pltpu.sync_copy(parts.at[0, pl.ds(off, CCH)], v0)
        pltpu.sync_copy(parts.at[1, pl.ds(off, CCH)], v1)

        def add_step(i, c2):
            sl = pl.ds(i * LANES, LANES)
            v0[sl] = v0[sl] + v1[sl]
            return c2

        lax.fori_loop(0, CCH // LANES, add_step, 0)
        pltpu.sync_copy(v0, out.at[pl.ds(off, CCH)])
        return carry

    lax.fori_loop(0, FPW // CCH, step, 0)


def kernel(direct_effects, edge_index, W):
    x = direct_effects.astype(jnp.float32)
    npad_e = E_PAD - N_EDGES
    src = jnp.concatenate(
        [edge_index[0].astype(jnp.int32), jnp.zeros((npad_e,), jnp.int32)])
    dst = jnp.concatenate(
        [edge_index[1].astype(jnp.int32),
         jnp.full((npad_e,), N_PAD - 1, jnp.int32)])
    wf = jnp.concatenate(
        [W.astype(jnp.float32), jnp.zeros((npad_e,), jnp.float32)])

    src = src.reshape(NW, NCHUNK, 1, CHUNK)
    dst = dst.reshape(NW, NCHUNK, 1, CHUNK)
    w3 = wf.reshape(NW, NCHUNK, CHUNK)
    wts = jnp.broadcast_to(w3[..., None], (NW, NCHUNK, CHUNK, LANES))
    wts = wts.reshape(NW, NCHUNK, 1, CHUNK * LANES)

    p = jnp.pad(x.T, ((0, N_PAD - N_GENES), (0, 0)))
    for _ in range(K_STEPS):
        parts = _scatter_step(p, src, dst, wts)
        p = _combine(parts.reshape(2, FLAT)).reshape(N_PAD, BATCH)
    return p[:N_GENES].T


# speedup vs baseline: 5.4897x; 1.0637x over previous
import functools

import jax
import jax.numpy as jnp
from jax import lax
from jax.experimental import pallas as pl
from jax.experimental.pallas import tpu as pltpu
from jax.experimental.pallas import tpu_sc as plsc

N_GENES = 10000
N_EDGES = 320000
BATCH = 128
K_STEPS = 3

N_PAD = 10240
NW = 32
CHUNK = 80
NCHUNK = 126
EPW = NCHUNK * CHUNK
E_PAD = NW * EPW
ROWS_PER_TILE = N_PAD // 16
INIT_CHUNK = 128
LANES = 16
VREGS_PER_ROW = BATCH // LANES
NSLOT = 3

FLAT = N_PAD * BATCH
FPW = FLAT // NW
CCH = 8192

_mesh = plsc.VectorSubcoreMesh(core_axis_name="c", subcore_axis_name="s")


@functools.partial(
    pl.kernel,
    out_type=jax.ShapeDtypeStruct((2, N_PAD, BATCH), jnp.float32),
    mesh=_mesh,
    scratch_types=[
        pltpu.VMEM_SHARED((N_PAD, BATCH), jnp.float32),
        pltpu.VMEM((NSLOT, 1, CHUNK), jnp.int32),
        pltpu.VMEM((NSLOT, 1, CHUNK), jnp.int32),
        pltpu.VMEM((NSLOT, 1, CHUNK * LANES), jnp.float32),
        pltpu.VMEM((NSLOT, CHUNK, BATCH), jnp.float32),
        pltpu.SemaphoreType.DMA, pltpu.SemaphoreType.DMA, pltpu.SemaphoreType.DMA,
        pltpu.SemaphoreType.DMA, pltpu.SemaphoreType.DMA, pltpu.SemaphoreType.DMA,
        pltpu.SemaphoreType.DMA, pltpu.SemaphoreType.DMA, pltpu.SemaphoreType.DMA,
    ],
)
def _scatter_step(p_hbm, srcR, dstR, wR, out_hbm,
                  acc, srcb, dstb, wbuf, rows,
                  isem0, isem1, isem2, gsem0, gsem1, gsem2, ssem0, ssem1, ssem2):
    isem = (isem0, isem1, isem2)
    gsem = (gsem0, gsem1, gsem2)
    ssem = (ssem0, ssem1, ssem2)

    cid = lax.axis_index("c")
    sid = lax.axis_index("s")
    wid = cid * 16 + sid
    row0 = sid * ROWS_PER_TILE

    zero16f = jnp.zeros((LANES,), jnp.float32)

    @plsc.parallel_loop(0, CHUNK, unroll=4)
    def _(r):
        for j in range(VREGS_PER_ROW):
            rows[0, r, pl.ds(j * LANES, LANES)] = zero16f

    @pl.when(cid == 0)
    def _():
        def cp(k, c2):
            base = row0 + k * INIT_CHUNK
            pltpu.sync_copy(p_hbm.at[pl.ds(base, INIT_CHUNK)],
                            acc.at[pl.ds(base, INIT_CHUNK)])
            return c2

        lax.fori_loop(0, ROWS_PER_TILE // INIT_CHUNK, cp, 0)

    @pl.when(cid == 1)
    def _():
        def zf(k, c2):
            base = row0 + k * CHUNK
            pltpu.sync_copy(rows.at[0], acc.at[pl.ds(base, CHUNK)])
            return c2

        lax.fori_loop(0, ROWS_PER_TILE // CHUNK, zf, 0)

    plsc.subcore_barrier()

    def load_idx(ci, s):
        pltpu.async_copy(srcR.at[wid, ci], srcb.at[s], isem[s])

    def wait_idx(ci, s):
        pltpu.make_async_copy(srcR.at[wid, ci], srcb.at[s], isem[s]).wait()

    def issue_gather(ci, s):
        pltpu.async_copy(p_hbm.at[srcb.at[s, 0]], rows.at[s], gsem[s])
        pltpu.async_copy(wR.at[wid, ci], wbuf.at[s], gsem[s])
        pltpu.async_copy(dstR.at[wid, ci], dstb.at[s], gsem[s])

    def wait_gather(ci, s):
        pltpu.make_async_copy(p_hbm.at[srcb.at[s, 0]], rows.at[s], gsem[s]).wait()
        pltpu.make_async_copy(wR.at[wid, ci], wbuf.at[s], gsem[s]).wait()
        pltpu.make_async_copy(dstR.at[wid, ci], dstb.at[s], gsem[s]).wait()

    def issue_scatter(s):
        pltpu.async_copy(rows.at[s], acc.at[dstb.at[s, 0]], ssem[s], add=True)

    def wait_scatter(s):
        pltpu.make_async_copy(rows.at[s], acc.at[dstb.at[s, 0]], ssem[s]).wait()

    def multiply(s):
        @plsc.parallel_loop(0, CHUNK, unroll=4)
        def _(e):
            wb = wbuf[s, 0, pl.ds(e * LANES, LANES)]
            for j in range(VREGS_PER_ROW):
                sl = pl.ds(j * LANES, LANES)
                rows[s, e, sl] = rows[s, e, sl] * wb

    load_idx(0, 0)
    load_idx(1, 1)
    load_idx(2, 2)
    wait_idx(0, 0)
    issue_gather(0, 0)
    wait_idx(1, 1)
    issue_gather(1, 1)

    def group(g, carry):
        for b in range(NSLOT):
            ci = g * NSLOT + b
            wait_gather(ci, b)

            @pl.when(ci + NSLOT < NCHUNK)
            def _():
                load_idx(ci + NSLOT, b)

            multiply(b)
            issue_scatter(b)

            s2 = (b + 2) % NSLOT

            @pl.when(ci + 2 < NCHUNK)
            def _():
                wait_idx(ci + 2, s2)

                @pl.when(ci >= 1)
                def _():
                    wait_scatter(s2)

                issue_gather(ci + 2, s2)
        return carry

    lax.fori_loop(0, NCHUNK // NSLOT, group, 0)

    wait_scatter(0)
    wait_scatter(1)
    wait_scatter(2)

    plsc.subcore_barrier()
    pltpu.sync_copy(acc.at[pl.ds(row0, ROWS_PER_TILE)],
                    out_hbm.at[cid, pl.ds(row0, ROWS_PER_TILE)])


@functools.partial(
    pl.kernel,
    out_type=jax.ShapeDtypeStruct((FLAT,), jnp.float32),
    mesh=_mesh,
    scratch_types=[
        pltpu.VMEM((CCH,), jnp.float32),
        pltpu.VMEM((CCH,), jnp.float32),
    ],
)
def _combine(parts, out, v0, v1):
    cid = lax.axis_index("c")
    sid = lax.axis_index("s")
    wid = cid * 16 + sid
    base = wid * FPW

    def step(k, carry):
        off = base + k * CCH
        pltpu.sync_copy(parts.at[0, pl.ds(off, CCH)], v0)
        pltpu.sync_copy(parts.at[1, pl.ds(off, CCH)], v1)

        @plsc.parallel_loop(0, CCH // LANES, unroll=8)
        def _(i):
            sl = pl.ds(i * LANES, LANES)
            v0[sl] = v0[sl] + v1[sl]
        pltpu.sync_copy(v0, out.at[pl.ds(off, CCH)])
        return carry

    lax.fori_loop(0, FPW // CCH, step, 0)


def kernel(direct_effects, edge_index, W):
    x = direct_effects.astype(jnp.float32)
    npad_e = E_PAD - N_EDGES
    src = jnp.concatenate(
        [edge_index[0].astype(jnp.int32), jnp.zeros((npad_e,), jnp.int32)])
    dst = jnp.concatenate(
        [edge_index[1].astype(jnp.int32),
         jnp.full((npad_e,), N_PAD - 1, jnp.int32)])
    wf = jnp.concatenate(
        [W.astype(jnp.float32), jnp.zeros((npad_e,), jnp.float32)])

    src = src.reshape(NW, NCHUNK, 1, CHUNK)
    dst = dst.reshape(NW, NCHUNK, 1, CHUNK)
    w3 = wf.reshape(NW, NCHUNK, CHUNK)
    wts = jnp.broadcast_to(w3[..., None], (NW, NCHUNK, CHUNK, LANES))
    wts = wts.reshape(NW, NCHUNK, 1, CHUNK * LANES)

    p = jnp.pad(x.T, ((0, N_PAD - N_GENES), (0, 0)))
    for _ in range(K_STEPS):
        parts = _scatter_step(p, src, dst, wts)
        p = _combine(parts.reshape(2, FLAT)).reshape(N_PAD, BATCH)
    return p[:N_GENES].T


# speedup vs baseline: 5.9556x; 1.0849x over previous
import functools

import jax
import jax.numpy as jnp
from jax import lax
from jax.experimental import pallas as pl
from jax.experimental.pallas import tpu as pltpu
from jax.experimental.pallas import tpu_sc as plsc

N_GENES = 10000
N_EDGES = 320000
BATCH = 128
K_STEPS = 3

N_PAD = 10240
NW = 32
CHUNK = 80
NCHUNK = 126
EPW = NCHUNK * CHUNK
E_PAD = NW * EPW
ROWS_PER_TILE = N_PAD // 16
INIT_CHUNK = 128
LANES = 16
VREGS_PER_ROW = BATCH // LANES
NSLOT = 3

_mesh = plsc.VectorSubcoreMesh(core_axis_name="c", subcore_axis_name="s")


@functools.partial(
    pl.kernel,
    out_type=(
        jax.ShapeDtypeStruct((N_PAD, BATCH), jnp.float32),
        jax.ShapeDtypeStruct((N_PAD, BATCH), jnp.float32),
        jax.ShapeDtypeStruct((N_PAD, BATCH), jnp.float32),
    ),
    mesh=_mesh,
    scratch_types=[
        pltpu.VMEM_SHARED((N_PAD, BATCH), jnp.float32),
        pltpu.VMEM((NSLOT, 1, CHUNK), jnp.int32),
        pltpu.VMEM((NSLOT, 1, CHUNK), jnp.int32),
        pltpu.VMEM((NSLOT, 1, CHUNK * LANES), jnp.float32),
        pltpu.VMEM((NSLOT, CHUNK, BATCH), jnp.float32),
        pltpu.VMEM((1, CHUNK), jnp.int32),
        pltpu.SemaphoreType.DMA, pltpu.SemaphoreType.DMA, pltpu.SemaphoreType.DMA,
        pltpu.SemaphoreType.DMA, pltpu.SemaphoreType.DMA, pltpu.SemaphoreType.DMA,
        pltpu.SemaphoreType.DMA, pltpu.SemaphoreType.DMA, pltpu.SemaphoreType.DMA,
        pltpu.SemaphoreType.REGULAR,
    ],
)
def _neumann(p0_hbm, srcR, dstR, wR, pout_hbm, pwork_hbm, a1_hbm,
             acc, srcb, dstb, wbuf, rows, idbuf,
             isem0, isem1, isem2, gsem0, gsem1, gsem2, ssem0, ssem1, ssem2,
             csem):
    isem = (isem0, isem1, isem2)
    gsem = (gsem0, gsem1, gsem2)
    ssem = (ssem0, ssem1, ssem2)

    cid = lax.axis_index("c")
    sid = lax.axis_index("s")
    wid = cid * 16 + sid
    row0 = sid * ROWS_PER_TILE

    zero16f = jnp.zeros((LANES,), jnp.float32)
    iota16 = lax.iota(jnp.int32, LANES)

    def zero_acc():
        @plsc.parallel_loop(0, CHUNK, unroll=4)
        def _(r):
            for j in range(VREGS_PER_ROW):
                rows[0, r, pl.ds(j * LANES, LANES)] = zero16f

        def zf(k, c2):
            base = row0 + k * CHUNK
            pltpu.sync_copy(rows.at[0], acc.at[pl.ds(base, CHUNK)])
            return c2

        lax.fori_loop(0, ROWS_PER_TILE // CHUNK, zf, 0)

    def seed_acc(p_hbm):
        def cp(k, c2):
            base = row0 + k * INIT_CHUNK
            pltpu.sync_copy(p_hbm.at[pl.ds(base, INIT_CHUNK)],
                            acc.at[pl.ds(base, INIT_CHUNK)])
            return c2

        lax.fori_loop(0, ROWS_PER_TILE // INIT_CHUNK, cp, 0)

    def load_idx(ci, s):
        pltpu.async_copy(srcR.at[wid, ci], srcb.at[s], isem[s])

    def wait_idx(ci, s):
        pltpu.make_async_copy(srcR.at[wid, ci], srcb.at[s], isem[s]).wait()

    def issue_gather(p_hbm, ci, s):
        pltpu.async_copy(p_hbm.at[srcb.at[s, 0]], rows.at[s], gsem[s])
        pltpu.async_copy(wR.at[wid, ci], wbuf.at[s], gsem[s])
        pltpu.async_copy(dstR.at[wid, ci], dstb.at[s], gsem[s])

    def wait_gather(p_hbm, ci, s):
        pltpu.make_async_copy(p_hbm.at[srcb.at[s, 0]], rows.at[s], gsem[s]).wait()
        pltpu.make_async_copy(wR.at[wid, ci], wbuf.at[s], gsem[s]).wait()
        pltpu.make_async_copy(dstR.at[wid, ci], dstb.at[s], gsem[s]).wait()

    def issue_scatter(s):
        pltpu.async_copy(rows.at[s], acc.at[dstb.at[s, 0]], ssem[s], add=True)

    def wait_scatter(s):
        pltpu.make_async_copy(rows.at[s], acc.at[dstb.at[s, 0]], ssem[s]).wait()

    def multiply(s):
        @plsc.parallel_loop(0, CHUNK, unroll=4)
        def _(e):
            wb = wbuf[s, 0, pl.ds(e * LANES, LANES)]
            for j in range(VREGS_PER_ROW):
                sl = pl.ds(j * LANES, LANES)
                rows[s, e, sl] = rows[s, e, sl] * wb

    def edge_pass(p_hbm):
        load_idx(0, 0)
        load_idx(1, 1)
        load_idx(2, 2)
        wait_idx(0, 0)
        issue_gather(p_hbm, 0, 0)
        wait_idx(1, 1)
        issue_gather(p_hbm, 1, 1)

        def group(g, carry):
            for b in range(NSLOT):
                ci = g * NSLOT + b
                wait_gather(p_hbm, ci, b)

                @pl.when(ci + NSLOT < NCHUNK)
                def _():
                    load_idx(ci + NSLOT, b)

                multiply(b)
                issue_scatter(b)

                s2 = (b + 2) % NSLOT

                @pl.when(ci + 2 < NCHUNK)
                def _():
                    wait_idx(ci + 2, s2)

                    @pl.when(ci >= 1)
                    def _():
                        wait_scatter(s2)

                    issue_gather(p_hbm, ci + 2, s2)
            return carry

        lax.fori_loop(0, NCHUNK // NSLOT, group, 0)
        wait_scatter(0)
        wait_scatter(1)
        wait_scatter(2)

    def combine_into(pnew_hbm):
        @pl.when(cid == 1)
        def _():
            def pub(k, c2):
                base = row0 + k * INIT_CHUNK
                pltpu.sync_copy(acc.at[pl.ds(base, INIT_CHUNK)],
                                a1_hbm.at[pl.ds(base, INIT_CHUNK)])
                return c2

            lax.fori_loop(0, ROWS_PER_TILE // INIT_CHUNK, pub, 0)

        plsc.subcore_barrier()
        pltpu.core_barrier(csem, core_axis_name="c")

        @pl.when(cid == 0)
        def _():
            def fold(k, c2):
                base = row0 + k * CHUNK
                pltpu.sync_copy(a1_hbm.at[pl.ds(base, CHUNK)], rows.at[1])
                for v in range(CHUNK // LANES):
                    idbuf[0, pl.ds(v * LANES, LANES)] = (
                        iota16 + (base + v * LANES))
                pltpu.sync_copy(rows.at[1],
                                acc.at[idbuf.at[0]], add=True)
                pltpu.sync_copy(acc.at[pl.ds(base, CHUNK)],
                                pnew_hbm.at[pl.ds(base, CHUNK)])
                return c2

            lax.fori_loop(0, ROWS_PER_TILE // CHUNK, fold, 0)

        @pl.when(cid == 1)
        def _():
            zero_acc()

        plsc.subcore_barrier()
        pltpu.core_barrier(csem, core_axis_name="c")

    @pl.when(cid == 0)
    def _():
        seed_acc(p0_hbm)

    @pl.when(cid == 1)
    def _():
        zero_acc()

    plsc.subcore_barrier()

    edge_pass(p0_hbm)
    plsc.subcore_barrier()
    combine_into(pwork_hbm)
    plsc.subcore_barrier()

    edge_pass(pwork_hbm)
    plsc.subcore_barrier()
    combine_into(pwork_hbm)
    plsc.subcore_barrier()

    edge_pass(pwork_hbm)
    plsc.subcore_barrier()
    combine_into(pout_hbm)


def kernel(direct_effects, edge_index, W):
    x = direct_effects.astype(jnp.float32)
    npad_e = E_PAD - N_EDGES
    src = jnp.concatenate(
        [edge_index[0].astype(jnp.int32), jnp.zeros((npad_e,), jnp.int32)])
    dst = jnp.concatenate(
        [edge_index[1].astype(jnp.int32),
         jnp.full((npad_e,), N_PAD - 1, jnp.int32)])
    wf = jnp.concatenate(
        [W.astype(jnp.float32), jnp.zeros((npad_e,), jnp.float32)])

    src = src.reshape(NW, NCHUNK, 1, CHUNK)
    dst = dst.reshape(NW, NCHUNK, 1, CHUNK)
    w3 = wf.reshape(NW, NCHUNK, CHUNK)
    wts = jnp.broadcast_to(w3[..., None], (NW, NCHUNK, CHUNK, LANES))
    wts = wts.reshape(NW, NCHUNK, 1, CHUNK * LANES)

    p = jnp.pad(x.T, ((0, N_PAD - N_GENES), (0, 0)))
    pfin, _, _ = _neumann(p, src, dst, wts)
    return pfin[:N_GENES].T


# speedup vs baseline: 6.0752x; 1.0201x over previous
import functools

import jax
import jax.numpy as jnp
from jax import lax
from jax.experimental import pallas as pl
from jax.experimental.pallas import tpu as pltpu
from jax.experimental.pallas import tpu_sc as plsc

N_GENES = 10000
N_EDGES = 320000
BATCH = 128
K_STEPS = 3

N_PAD = 10240
NW = 32
CHUNK = 80
NCHUNK = 126
EPW = NCHUNK * CHUNK
E_PAD = NW * EPW
ROWS_PER_TILE = N_PAD // 16
INIT_CHUNK = 128
HALF = N_PAD // 2
FOLD_PER_TILE = HALF // 16
FCH = 80
LANES = 16
VREGS_PER_ROW = BATCH // LANES
NSLOT = 3

_mesh = plsc.VectorSubcoreMesh(core_axis_name="c", subcore_axis_name="s")


@functools.partial(
    pl.kernel,
    out_type=(
        jax.ShapeDtypeStruct((N_PAD, BATCH), jnp.float32),
        jax.ShapeDtypeStruct((N_PAD, BATCH), jnp.float32),
        jax.ShapeDtypeStruct((N_PAD, BATCH), jnp.float32),
    ),
    mesh=_mesh,
    scratch_types=[
        pltpu.VMEM_SHARED((N_PAD, BATCH), jnp.float32),
        pltpu.VMEM((NSLOT, 1, CHUNK), jnp.int32),
        pltpu.VMEM((NSLOT, 1, CHUNK), jnp.int32),
        pltpu.VMEM((NSLOT, 1, CHUNK * LANES), jnp.float32),
        pltpu.VMEM((NSLOT, CHUNK, BATCH), jnp.float32),
        pltpu.VMEM((1, CHUNK), jnp.int32),
        pltpu.SemaphoreType.DMA, pltpu.SemaphoreType.DMA, pltpu.SemaphoreType.DMA,
        pltpu.SemaphoreType.DMA, pltpu.SemaphoreType.DMA, pltpu.SemaphoreType.DMA,
        pltpu.SemaphoreType.DMA, pltpu.SemaphoreType.DMA, pltpu.SemaphoreType.DMA,
        pltpu.SemaphoreType.REGULAR,
    ],
)
def _neumann(p0_hbm, srcR, dstR, wR, pout_hbm, pwork_hbm, a1_hbm,
             acc, srcb, dstb, wbuf, rows, idbuf,
             isem0, isem1, isem2, gsem0, gsem1, gsem2, ssem0, ssem1, ssem2,
             csem):
    isem = (isem0, isem1, isem2)
    gsem = (gsem0, gsem1, gsem2)
    ssem = (ssem0, ssem1, ssem2)

    cid = lax.axis_index("c")
    sid = lax.axis_index("s")
    wid = cid * 16 + sid
    row0 = sid * ROWS_PER_TILE
    hb = cid * HALF + sid * FOLD_PER_TILE
    ob = (1 - cid) * HALF + sid * FOLD_PER_TILE

    zero16f = jnp.zeros((LANES,), jnp.float32)
    iota16 = lax.iota(jnp.int32, LANES)

    def zero_acc():
        @plsc.parallel_loop(0, CHUNK, unroll=4)
        def _(r):
            for j in range(VREGS_PER_ROW):
                rows[0, r, pl.ds(j * LANES, LANES)] = zero16f

        def zf(k, c2):
            base = row0 + k * CHUNK
            pltpu.sync_copy(rows.at[0], acc.at[pl.ds(base, CHUNK)])
            return c2

        lax.fori_loop(0, ROWS_PER_TILE // CHUNK, zf, 0)

    def seed_acc(p_hbm):
        def cp(k, c2):
            base = row0 + k * INIT_CHUNK
            pltpu.sync_copy(p_hbm.at[pl.ds(base, INIT_CHUNK)],
                            acc.at[pl.ds(base, INIT_CHUNK)])
            return c2

        lax.fori_loop(0, ROWS_PER_TILE // INIT_CHUNK, cp, 0)

    def load_idx(ci, s):
        pltpu.async_copy(srcR.at[wid, ci], srcb.at[s], isem[s])

    def wait_idx(ci, s):
        pltpu.make_async_copy(srcR.at[wid, ci], srcb.at[s], isem[s]).wait()

    def issue_gather(p_hbm, ci, s):
        pltpu.async_copy(p_hbm.at[srcb.at[s, 0]], rows.at[s], gsem[s])
        pltpu.async_copy(wR.at[wid, ci], wbuf.at[s], gsem[s])
        pltpu.async_copy(dstR.at[wid, ci], dstb.at[s], gsem[s])

    def wait_gather(p_hbm, ci, s):
        pltpu.make_async_copy(p_hbm.at[srcb.at[s, 0]], rows.at[s], gsem[s]).wait()
        pltpu.make_async_copy(wR.at[wid, ci], wbuf.at[s], gsem[s]).wait()
        pltpu.make_async_copy(dstR.at[wid, ci], dstb.at[s], gsem[s]).wait()

    def issue_scatter(s):
        pltpu.async_copy(rows.at[s], acc.at[dstb.at[s, 0]], ssem[s], add=True)

    def wait_scatter(s):
        pltpu.make_async_copy(rows.at[s], acc.at[dstb.at[s, 0]], ssem[s]).wait()

    def multiply(s):
        @plsc.parallel_loop(0, CHUNK, unroll=4)
        def _(e):
            wb = wbuf[s, 0, pl.ds(e * LANES, LANES)]
            for j in range(VREGS_PER_ROW):
                sl = pl.ds(j * LANES, LANES)
                rows[s, e, sl] = rows[s, e, sl] * wb

    def edge_pass(p_hbm):
        load_idx(0, 0)
        load_idx(1, 1)
        load_idx(2, 2)
        wait_idx(0, 0)
        issue_gather(p_hbm, 0, 0)
        wait_idx(1, 1)
        issue_gather(p_hbm, 1, 1)

        def group(g, carry):
            for b in range(NSLOT):
                ci = g * NSLOT + b
                wait_gather(p_hbm, ci, b)

                @pl.when(ci + NSLOT < NCHUNK)
                def _():
                    load_idx(ci + NSLOT, b)

                multiply(b)
                issue_scatter(b)

                s2 = (b + 2) % NSLOT

                @pl.when(ci + 2 < NCHUNK)
                def _():
                    wait_idx(ci + 2, s2)

                    @pl.when(ci >= 1)
                    def _():
                        wait_scatter(s2)

                    issue_gather(p_hbm, ci + 2, s2)
            return carry

        lax.fori_loop(0, NCHUNK // NSLOT, group, 0)
        wait_scatter(0)
        wait_scatter(1)
        wait_scatter(2)

    def combine_into(pnew_hbm):
        pltpu.sync_copy(acc.at[pl.ds(ob, FOLD_PER_TILE)],
                        a1_hbm.at[pl.ds(ob, FOLD_PER_TILE)])
        plsc.subcore_barrier()
        pltpu.core_barrier(csem, core_axis_name="c")

        def fold(k, c2):
            base = hb + k * FCH
            pltpu.sync_copy(a1_hbm.at[pl.ds(base, FCH)], rows.at[1])
            for v in range(FCH // LANES):
                idbuf[0, pl.ds(v * LANES, LANES)] = (
                    iota16 + (base + v * LANES))
            pltpu.sync_copy(rows.at[1], acc.at[idbuf.at[0]], add=True)
            pltpu.sync_copy(acc.at[pl.ds(base, FCH)],
                            pnew_hbm.at[pl.ds(base, FCH)])
            return c2

        lax.fori_loop(0, FOLD_PER_TILE // FCH, fold, 0)

        plsc.subcore_barrier()
        pltpu.core_barrier(csem, core_axis_name="c")

        @pl.when(cid == 0)
        def _():
            pltpu.sync_copy(pnew_hbm.at[pl.ds(ob, FOLD_PER_TILE)],
                            acc.at[pl.ds(ob, FOLD_PER_TILE)])

        @pl.when(cid == 1)
        def _():
            zero_acc()

        plsc.subcore_barrier()

    @pl.when(cid == 0)
    def _():
        seed_acc(p0_hbm)

    @pl.when(cid == 1)
    def _():
        zero_acc()

    plsc.subcore_barrier()

    edge_pass(p0_hbm)
    plsc.subcore_barrier()
    combine_into(pwork_hbm)
    plsc.subcore_barrier()

    edge_pass(pwork_hbm)
    plsc.subcore_barrier()
    combine_into(pwork_hbm)
    plsc.subcore_barrier()

    edge_pass(pwork_hbm)
    plsc.subcore_barrier()
    combine_into(pout_hbm)


def kernel(direct_effects, edge_index, W):
    x = direct_effects.astype(jnp.float32)
    npad_e = E_PAD - N_EDGES
    src = jnp.concatenate(
        [edge_index[0].astype(jnp.int32), jnp.zeros((npad_e,), jnp.int32)])
    dst = jnp.concatenate(
        [edge_index[1].astype(jnp.int32),
         jnp.full((npad_e,), N_PAD - 1, jnp.int32)])
    wf = jnp.concatenate(
        [W.astype(jnp.float32), jnp.zeros((npad_e,), jnp.float32)])

    src = src.reshape(NW, NCHUNK, 1, CHUNK)
    dst = dst.reshape(NW, NCHUNK, 1, CHUNK)
    w3 = wf.reshape(NW, NCHUNK, CHUNK)
    wts = jnp.broadcast_to(w3[..., None], (NW, NCHUNK, CHUNK, LANES))
    wts = wts.reshape(NW, NCHUNK, 1, CHUNK * LANES)

    p = jnp.pad(x.T, ((0, N_PAD - N_GENES), (0, 0)))
    pfin, _, _ = _neumann(p, src, dst, wts)
    return pfin[:N_GENES].T


# speedup vs baseline: 6.4789x; 1.0665x over previous
import functools

import jax
import jax.numpy as jnp
from jax import lax
from jax.experimental import pallas as pl
from jax.experimental.pallas import tpu as pltpu
from jax.experimental.pallas import tpu_sc as plsc

N_GENES = 10000
N_EDGES = 320000
BATCH = 128
K_STEPS = 3

N_PAD = 10240
NW = 32
CHUNK = 80
NCHUNK = 126
EPW = NCHUNK * CHUNK
E_PAD = NW * EPW
ROWS_PER_TILE = N_PAD // 16
INIT_CHUNK = 128
HALF = N_PAD // 2
FOLD_PER_TILE = HALF // 16
FCH = 80
LANES = 16
VREGS_PER_ROW = BATCH // LANES
NSLOT = 3

_mesh = plsc.VectorSubcoreMesh(core_axis_name="c", subcore_axis_name="s")


@functools.partial(
    pl.kernel,
    out_type=(
        jax.ShapeDtypeStruct((N_PAD, BATCH), jnp.float32),
        jax.ShapeDtypeStruct((N_PAD, BATCH), jnp.float32),
        jax.ShapeDtypeStruct((N_PAD, BATCH), jnp.float32),
    ),
    mesh=_mesh,
    scratch_types=[
        pltpu.VMEM_SHARED((N_PAD, BATCH), jnp.float32),
        pltpu.VMEM((NSLOT, 1, CHUNK), jnp.int32),
        pltpu.VMEM((NSLOT, 1, CHUNK), jnp.int32),
        pltpu.VMEM((NSLOT, 1, CHUNK), jnp.float32),
        pltpu.VMEM((NSLOT, CHUNK, BATCH), jnp.float32),
        pltpu.VMEM((1, CHUNK), jnp.int32),
        pltpu.SemaphoreType.DMA, pltpu.SemaphoreType.DMA, pltpu.SemaphoreType.DMA,
        pltpu.SemaphoreType.DMA, pltpu.SemaphoreType.DMA, pltpu.SemaphoreType.DMA,
        pltpu.SemaphoreType.DMA, pltpu.SemaphoreType.DMA, pltpu.SemaphoreType.DMA,
        pltpu.SemaphoreType.REGULAR,
    ],
)
def _neumann(p0_hbm, srcR, dstR, wR, pout_hbm, pwork_hbm, a1_hbm,
             acc, srcb, dstb, wbuf, rows, idbuf,
             isem0, isem1, isem2, gsem0, gsem1, gsem2, ssem0, ssem1, ssem2,
             csem):
    isem = (isem0, isem1, isem2)
    gsem = (gsem0, gsem1, gsem2)
    ssem = (ssem0, ssem1, ssem2)

    cid = lax.axis_index("c")
    sid = lax.axis_index("s")
    wid = cid * 16 + sid
    row0 = sid * ROWS_PER_TILE
    hb = cid * HALF + sid * FOLD_PER_TILE
    ob = (1 - cid) * HALF + sid * FOLD_PER_TILE

    zero16f = jnp.zeros((LANES,), jnp.float32)
    iota16 = lax.iota(jnp.int32, LANES)

    def zero_acc():
        @plsc.parallel_loop(0, CHUNK, unroll=4)
        def _(r):
            for j in range(VREGS_PER_ROW):
                rows[0, r, pl.ds(j * LANES, LANES)] = zero16f

        def zf(k, c2):
            base = row0 + k * CHUNK
            pltpu.sync_copy(rows.at[0], acc.at[pl.ds(base, CHUNK)])
            return c2

        lax.fori_loop(0, ROWS_PER_TILE // CHUNK, zf, 0)

    def seed_acc(p_hbm):
        def cp(k, c2):
            base = row0 + k * INIT_CHUNK
            pltpu.sync_copy(p_hbm.at[pl.ds(base, INIT_CHUNK)],
                            acc.at[pl.ds(base, INIT_CHUNK)])
            return c2

        lax.fori_loop(0, ROWS_PER_TILE // INIT_CHUNK, cp, 0)

    def load_idx(ci, s):
        pltpu.async_copy(srcR.at[wid, ci], srcb.at[s], isem[s])

    def wait_idx(ci, s):
        pltpu.make_async_copy(srcR.at[wid, ci], srcb.at[s], isem[s]).wait()

    def issue_gather(p_hbm, ci, s):
        pltpu.async_copy(p_hbm.at[srcb.at[s, 0]], rows.at[s], gsem[s])
        pltpu.async_copy(wR.at[wid, ci], wbuf.at[s], gsem[s])
        pltpu.async_copy(dstR.at[wid, ci], dstb.at[s], gsem[s])

    def wait_gather(p_hbm, ci, s):
        pltpu.make_async_copy(p_hbm.at[srcb.at[s, 0]], rows.at[s], gsem[s]).wait()
        pltpu.make_async_copy(wR.at[wid, ci], wbuf.at[s], gsem[s]).wait()
        pltpu.make_async_copy(dstR.at[wid, ci], dstb.at[s], gsem[s]).wait()

    def issue_scatter(s):
        pltpu.async_copy(rows.at[s], acc.at[dstb.at[s, 0]], ssem[s], add=True)

    def wait_scatter(s):
        pltpu.make_async_copy(rows.at[s], acc.at[dstb.at[s, 0]], ssem[s]).wait()

    def multiply(s):
        @plsc.parallel_loop(0, CHUNK // LANES)
        def _(g):
            w16 = wbuf[s, 0, pl.ds(g * LANES, LANES)]
            for lane in range(LANES):
                wb = w16[jnp.full((LANES,), lane, jnp.int32)]
                e = g * LANES + lane
                for j in range(VREGS_PER_ROW):
                    sl = pl.ds(j * LANES, LANES)
                    rows[s, e, sl] = rows[s, e, sl] * wb

    def edge_pass(p_hbm):
        load_idx(0, 0)
        load_idx(1, 1)
        load_idx(2, 2)
        wait_idx(0, 0)
        issue_gather(p_hbm, 0, 0)
        wait_idx(1, 1)
        issue_gather(p_hbm, 1, 1)

        def group(g, carry):
            for b in range(NSLOT):
                ci = g * NSLOT + b
                wait_gather(p_hbm, ci, b)

                @pl.when(ci + NSLOT < NCHUNK)
                def _():
                    load_idx(ci + NSLOT, b)

                multiply(b)
                issue_scatter(b)

                s2 = (b + 2) % NSLOT

                @pl.when(ci + 2 < NCHUNK)
                def _():
                    wait_idx(ci + 2, s2)

                    @pl.when(ci >= 1)
                    def _():
                        wait_scatter(s2)

                    issue_gather(p_hbm, ci + 2, s2)
            return carry

        lax.fori_loop(0, NCHUNK // NSLOT, group, 0)
        wait_scatter(0)
        wait_scatter(1)
        wait_scatter(2)

    def combine_into(pnew_hbm):
        pltpu.sync_copy(acc.at[pl.ds(ob, FOLD_PER_TILE)],
                        a1_hbm.at[pl.ds(ob, FOLD_PER_TILE)])
        plsc.subcore_barrier()
        pltpu.core_barrier(csem, core_axis_name="c")

        def fold(k, c2):
            base = hb + k * FCH
            pltpu.sync_copy(a1_hbm.at[pl.ds(base, FCH)], rows.at[1])
            for v in range(FCH // LANES):
                idbuf[0, pl.ds(v * LANES, LANES)] = (
                    iota16 + (base + v * LANES))
            pltpu.sync_copy(rows.at[1], acc.at[idbuf.at[0]], add=True)
            pltpu.sync_copy(acc.at[pl.ds(base, FCH)],
                            pnew_hbm.at[pl.ds(base, FCH)])
            return c2

        lax.fori_loop(0, FOLD_PER_TILE // FCH, fold, 0)

        plsc.subcore_barrier()
        pltpu.core_barrier(csem, core_axis_name="c")

        @pl.when(cid == 0)
        def _():
            pltpu.sync_copy(pnew_hbm.at[pl.ds(ob, FOLD_PER_TILE)],
                            acc.at[pl.ds(ob, FOLD_PER_TILE)])

        @pl.when(cid == 1)
        def _():
            zero_acc()

        plsc.subcore_barrier()

    @pl.when(cid == 0)
    def _():
        seed_acc(p0_hbm)

    @pl.when(cid == 1)
    def _():
        zero_acc()

    plsc.subcore_barrier()

    edge_pass(p0_hbm)
    plsc.subcore_barrier()
    combine_into(pwork_hbm)
    plsc.subcore_barrier()

    edge_pass(pwork_hbm)
    plsc.subcore_barrier()
    combine_into(pwork_hbm)
    plsc.subcore_barrier()

    edge_pass(pwork_hbm)
    plsc.subcore_barrier()
    combine_into(pout_hbm)


def kernel(direct_effects, edge_index, W):
    x = direct_effects.astype(jnp.float32)
    npad_e = E_PAD - N_EDGES
    src = jnp.concatenate(
        [edge_index[0].astype(jnp.int32), jnp.zeros((npad_e,), jnp.int32)])
    dst = jnp.concatenate(
        [edge_index[1].astype(jnp.int32),
         jnp.full((npad_e,), N_PAD - 1, jnp.int32)])
    wf = jnp.concatenate(
        [W.astype(jnp.float32), jnp.zeros((npad_e,), jnp.float32)])

    src = src.reshape(NW, NCHUNK, 1, CHUNK)
    dst = dst.reshape(NW, NCHUNK, 1, CHUNK)
    wts = wf.reshape(NW, NCHUNK, 1, CHUNK)

    p = jnp.pad(x.T, ((0, N_PAD - N_GENES), (0, 0)))
    pfin, _, _ = _neumann(p, src, dst, wts)
    return pfin[:N_GENES].T
